# baseline (speedup 1.0000x reference)
import functools

import jax
import jax.numpy as jnp
from jax import lax
from jax.experimental import pallas as pl
from jax.experimental.pallas import tpu as pltpu
from jax.experimental.pallas import tpu_sc as plsc

_B, _S, _D, _FF, _E, _TOP_K = 1, 2048, 768, 2048, 8, 2
_N = _B * _S
_T = 512
_NB = (4096 + _E * (_T - 1) + _T - 1) // _T
_PMAX = _NB * _T
_NC, _NS, _L = 2, 16, 16
_NW = _NC * _NS
_RPT = _PMAX // _NW
_GCH = 64
_TPT = _N // _NW
_DI = _D // 2
_PPS = 2 * _N // _NS
_IPS = _PMAX // _NS


def _router_body(x_ref, gw_ref, destp_ref, wp_ref, gid_ref, nb_ref, aux_ref):
    xf = x_ref[...]
    gw = gw_ref[...]
    lg = jax.lax.dot_general(gw, xf, (((1,), (1,)), ((), ())),
                             preferred_element_type=jnp.float32)
    eio = jax.lax.broadcasted_iota(jnp.int32, lg.shape, 0)
    m1 = jnp.max(lg, axis=0, keepdims=True)
    i1 = jnp.min(jnp.where(lg == m1, eio, _E), axis=0, keepdims=True)
    sel1 = eio == i1
    masked = jnp.where(sel1, -jnp.inf, lg)
    m2 = jnp.max(masked, axis=0, keepdims=True)
    i2 = jnp.min(jnp.where(masked == m2, eio, _E), axis=0, keepdims=True)
    sel2 = eio == i2
    w1 = 1.0 / (1.0 + jnp.exp(m2 - m1))
    w2 = 1.0 - w1
    ez = jnp.exp(lg - m1)
    probs = ez / jnp.sum(ez, axis=0, keepdims=True)
    pmean = jnp.sum(probs, axis=1, keepdims=True) / _N
    cnt = sel1.astype(jnp.float32) + sel2.astype(jnp.float32)
    counts = jnp.sum(cnt, axis=1, keepdims=True)
    aux_ref[0, 0] = _E * jnp.sum((counts / _N) * pmean)
    acc = cnt
    d = 1
    while d < _N:
        z = jnp.zeros((_E, d), jnp.float32)
        acc = acc + jnp.concatenate([z, acc[:, :-d]], axis=1)
        d *= 2
    prefix = acc - cnt
    cpad = jnp.floor((counts + (_T - 1)) / _T) * _T
    o = cpad
    o = o + jnp.concatenate([jnp.zeros((1, 1), jnp.float32), o[:-1]], axis=0)
    o = o + jnp.concatenate([jnp.zeros((2, 1), jnp.float32), o[:-2]], axis=0)
    o = o + jnp.concatenate([jnp.zeros((4, 1), jnp.float32), o[:-4]], axis=0)
    off = o - cpad
    end = off + cpad
    slot = off + prefix
    d1 = jnp.sum(jnp.where(sel1, slot, 0.0), axis=0, keepdims=True)
    d2 = jnp.sum(jnp.where(sel2, slot, 0.0), axis=0, keepdims=True)
    destp_ref[...] = jnp.concatenate([d1, d2], axis=0).astype(jnp.int32)
    wp_ref[...] = jnp.concatenate([w1, w2], axis=0)
    sb = jax.lax.broadcasted_iota(jnp.int32, (1, 64), 1).astype(
        jnp.float32) * _T
    g = jnp.sum((sb >= end).astype(jnp.float32), axis=0, keepdims=True)
    gid_ref[...] = jnp.minimum(g, _E - 1).astype(jnp.int32)
    nb_ref[0, 0] = (jnp.sum(cpad) / _T).astype(jnp.int32)


def _sc_dispatch_body(destp_hbm, wp_hbm, x_hbm, xs_hbm, scale_hbm,
                      initv, tokv, idx128a, idx128b, wv256, src_sh, scale_sh,
                      idxv0, idxv1, idxv2, idxv3,
                      rows0, rows1, rows2, rows3, gsem, wsem):
    cid = lax.axis_index("c")
    sid = lax.axis_index("s")
    ibase = sid * _IPS
    for c in range(_IPS // _L):
        initv[pl.ds(c * _L, _L)] = (
            lax.iota(jnp.int32, _L) + (ibase + c * _L)) % _N
    pltpu.sync_copy(initv, src_sh.at[pl.ds(ibase, _IPS)])
    plsc.subcore_barrier()
    pbase = sid * _PPS
    la = pltpu.async_copy(destp_hbm.at[pl.ds(pbase, 128)], idx128a, gsem)
    lb = pltpu.async_copy(destp_hbm.at[pl.ds(pbase + 128, 128)], idx128b,
                          gsem)
    lw = pltpu.async_copy(wp_hbm.at[pl.ds(pbase, _PPS)], wv256, wsem)
    for t in range(_PPS // _L):
        tokv[pl.ds(t * _L, _L)] = (
            lax.iota(jnp.int32, _L) + (pbase + t * _L)) % _N
    la.wait()
    lb.wait()
    lw.wait()
    s1 = pltpu.async_copy(tokv.at[pl.ds(0, 128)], src_sh.at[idx128a], gsem)
    s2 = pltpu.async_copy(tokv.at[pl.ds(128, 128)], src_sh.at[idx128b], gsem)
    s3 = pltpu.async_copy(wv256.at[pl.ds(0, 128)], scale_sh.at[idx128a],
                          wsem)
    s4 = pltpu.async_copy(wv256.at[pl.ds(128, 128)], scale_sh.at[idx128b],
                          wsem)
    s1.wait()
    s2.wait()
    s3.wait()
    s4.wait()
    plsc.subcore_barrier()

    @pl.when(jnp.logical_and(sid == 0, cid == 0))
    def _():
        pltpu.sync_copy(scale_sh, scale_hbm)

    wid = sid * _NC + cid
    tbase = wid * _RPT
    idxs = (idxv0, idxv1, idxv2, idxv3)
    rows = (rows0, rows1, rows2, rows3)
    gs = []
    for j in range(_RPT // _GCH):
        pltpu.sync_copy(src_sh.at[pl.ds(tbase + j * _GCH, _GCH)], idxs[j])
        gs.append(pltpu.async_copy(x_hbm.at[idxs[j]], rows[j], gsem))
    wbs = []
    for j in range(_RPT // _GCH):
        gs[j].wait()
        wbs.append(pltpu.async_copy(
            rows[j], xs_hbm.at[pl.ds(tbase + j * _GCH, _GCH)], wsem))
    for wb in wbs:
        wb.wait()


def _ffn_body(gid_ref, nb_ref, xs_ref, sc_ref, wg_ref, wu_ref, wd_ref,
              ys_ref):
    @pl.when(pl.program_id(0) < nb_ref[0])
    def _():
        xb = xs_ref[...].astype(jnp.float32)
        g = jax.lax.dot_general(xb, wg_ref[0], (((1,), (1,)), ((), ())),
                                preferred_element_type=jnp.float32)
        u = jax.lax.dot_general(xb, wu_ref[0], (((1,), (1,)), ((), ())),
                                preferred_element_type=jnp.float32)
        h = (g * jax.nn.sigmoid(g)) * u
        y = jax.lax.dot_general(h, wd_ref[0], (((1,), (1,)), ((), ())),
                                preferred_element_type=jnp.float32)
        ys_ref[...] = sc_ref[...] * y


def _sc_combine_body(d0_hbm, d1_hbm, ys_hbm, out_hbm,
                     idxa, idxb, ra, rb, sem):
    cid = lax.axis_index("c")
    sid = lax.axis_index("s")
    wid = sid * _NC + cid
    base = wid * _TPT
    pltpu.sync_copy(d0_hbm.at[pl.ds(base, _TPT)], idxa)
    pltpu.sync_copy(d1_hbm.at[pl.ds(base, _TPT)], idxb)
    ca = pltpu.async_copy(ys_hbm.at[idxa], ra, sem)
    cb = pltpu.async_copy(ys_hbm.at[idxb], rb, sem)
    ca.wait()
    cb.wait()

    def rowbody(r, _):
        for c in range(_D // _L):
            sl = pl.ds(c * _L, _L)
            ra[r, sl] = ra[r, sl] + rb[r, sl]
        return 0

    half = _TPT // 2
    lax.fori_loop(0, half, rowbody, 0)
    wb = pltpu.async_copy(ra.at[pl.ds(0, half)],
                          out_hbm.at[pl.ds(base, half)], sem)
    lax.fori_loop(half, _TPT, rowbody, 0)
    pltpu.sync_copy(ra.at[pl.ds(half, half)],
                    out_hbm.at[pl.ds(base + half, half)])
    wb.wait()


def _sc_dispatch(destp_flat, wp_flat, flat):
    wrapped = functools.partial(
        pl.kernel,
        out_type=(
            jax.ShapeDtypeStruct((_PMAX, _DI), jnp.int32),
            jax.ShapeDtypeStruct((_PMAX,), jnp.float32),
        ),
        mesh=plsc.VectorSubcoreMesh(core_axis_name="c", subcore_axis_name="s",
                                    num_cores=_NC, num_subcores=_NS),
        scratch_types=[
            pltpu.VMEM((_IPS,), jnp.int32),
            pltpu.VMEM((_PPS,), jnp.int32),
            pltpu.VMEM((128,), jnp.int32),
            pltpu.VMEM((128,), jnp.int32),
            pltpu.VMEM((_PPS,), jnp.float32),
            pltpu.VMEM_SHARED((_PMAX,), jnp.int32),
            pltpu.VMEM_SHARED((_PMAX,), jnp.float32),
            pltpu.VMEM((_GCH,), jnp.int32),
            pltpu.VMEM((_GCH,), jnp.int32),
            pltpu.VMEM((_GCH,), jnp.int32),
            pltpu.VMEM((_GCH,), jnp.int32),
            pltpu.VMEM((_GCH, _DI), jnp.int32),
            pltpu.VMEM((_GCH, _DI), jnp.int32),
            pltpu.VMEM((_GCH, _DI), jnp.int32),
            pltpu.VMEM((_GCH, _DI), jnp.int32),
            pltpu.SemaphoreType.DMA,
            pltpu.SemaphoreType.DMA,
        ],
        compiler_params=pltpu.CompilerParams(needs_layout_passes=False),
    )(_sc_dispatch_body)
    return wrapped(destp_flat, wp_flat, flat)


def _sc_combine(d0, d1, ys):
    wrapped = functools.partial(
        pl.kernel,
        out_type=jax.ShapeDtypeStruct((_N, _D), jnp.float32),
        mesh=plsc.VectorSubcoreMesh(core_axis_name="c", subcore_axis_name="s",
                                    num_cores=_NC, num_subcores=_NS),
        scratch_types=[
            pltpu.VMEM((_TPT,), jnp.int32),
            pltpu.VMEM((_TPT,), jnp.int32),
            pltpu.VMEM((_TPT, _D), jnp.float32),
            pltpu.VMEM((_TPT, _D), jnp.float32),
            pltpu.SemaphoreType.DMA,
        ],
        compiler_params=pltpu.CompilerParams(needs_layout_passes=False),
    )(_sc_combine_body)
    return wrapped(d0, d1, ys)


@jax.jit
def kernel(x, gate_w, w_gate, w_up, w_down):
    flat = x.reshape(_N, _D)

    destp, wp, gid, nb, aux = pl.pallas_call(
        _router_body,
        out_shape=(
            jax.ShapeDtypeStruct((2, _N), jnp.int32),
            jax.ShapeDtypeStruct((2, _N), jnp.float32),
            jax.ShapeDtypeStruct((1, 64), jnp.int32),
            jax.ShapeDtypeStruct((1, 1), jnp.int32),
            jax.ShapeDtypeStruct((1, 1), jnp.float32),
        ),
        in_specs=[
            pl.BlockSpec(memory_space=pltpu.VMEM),
            pl.BlockSpec(memory_space=pltpu.VMEM),
        ],
        out_specs=(
            pl.BlockSpec(memory_space=pltpu.VMEM),
            pl.BlockSpec(memory_space=pltpu.VMEM),
            pl.BlockSpec(memory_space=pltpu.VMEM),
            pl.BlockSpec(memory_space=pltpu.SMEM),
            pl.BlockSpec(memory_space=pltpu.SMEM),
        ),
    )(flat, gate_w)

    x32 = jax.lax.bitcast_convert_type(
        flat.astype(jnp.bfloat16).reshape(_N, _DI, 2), jnp.int32)
    xs32, scale = _sc_dispatch(destp.reshape(2 * _N), wp.reshape(2 * _N),
                               x32)
    xs = jax.lax.bitcast_convert_type(xs32, jnp.bfloat16).reshape(_PMAX, _D)
    scale = scale.reshape(_PMAX, 1)

    ys = pl.pallas_call(
        _ffn_body,
        grid_spec=pltpu.PrefetchScalarGridSpec(
            num_scalar_prefetch=2,
            grid=(_NB,),
            in_specs=[
                pl.BlockSpec((_T, _D), lambda b, gid, nb: (b, 0)),
                pl.BlockSpec((_T, 1), lambda b, gid, nb: (b, 0)),
                pl.BlockSpec((1, _FF, _D), lambda b, gid, nb: (gid[b], 0, 0)),
                pl.BlockSpec((1, _FF, _D), lambda b, gid, nb: (gid[b], 0, 0)),
                pl.BlockSpec((1, _D, _FF), lambda b, gid, nb: (gid[b], 0, 0)),
            ],
            out_specs=pl.BlockSpec((_T, _D), lambda b, gid, nb: (b, 0)),
        ),
        out_shape=jax.ShapeDtypeStruct((_PMAX, _D), jnp.float32),
    )(gid.reshape(64), nb.reshape(1), xs, scale, w_gate, w_up, w_down)

    out = _sc_combine(destp[0], destp[1], ys)
    return out.reshape(_B, _S, _D), aux.reshape(())


# speedup vs baseline: 1.7669x; 1.7669x over previous
import functools

import jax
import jax.numpy as jnp
from jax import lax
from jax.experimental import pallas as pl
from jax.experimental.pallas import tpu as pltpu
from jax.experimental.pallas import tpu_sc as plsc

_B, _S, _D, _FF, _E, _TOP_K = 1, 2048, 768, 2048, 8, 2
_N = _B * _S
_T = 512
_NB = (4096 + _E * (_T - 1) + _T - 1) // _T
_PMAX = _NB * _T
_NC, _NS, _L = 2, 16, 16
_NW = _NC * _NS
_RPT = _PMAX // _NW
_GCH = 64
_TPT = _N // _NW
_DI = _D // 2
_PPS = 2 * _N // _NS
_IPS = _PMAX // _NS


def _router_body(x_ref, gw_ref, destp_ref, wp_ref, gid_ref, nb_ref, aux_ref):
    xf = x_ref[...]
    gw = gw_ref[...]
    lg = jax.lax.dot_general(gw, xf, (((1,), (1,)), ((), ())),
                             preferred_element_type=jnp.float32)
    eio = jax.lax.broadcasted_iota(jnp.int32, lg.shape, 0)
    m1 = jnp.max(lg, axis=0, keepdims=True)
    i1 = jnp.min(jnp.where(lg == m1, eio, _E), axis=0, keepdims=True)
    sel1 = eio == i1
    masked = jnp.where(sel1, -jnp.inf, lg)
    m2 = jnp.max(masked, axis=0, keepdims=True)
    i2 = jnp.min(jnp.where(masked == m2, eio, _E), axis=0, keepdims=True)
    sel2 = eio == i2
    w1 = 1.0 / (1.0 + jnp.exp(m2 - m1))
    w2 = 1.0 - w1
    ez = jnp.exp(lg - m1)
    probs = ez / jnp.sum(ez, axis=0, keepdims=True)
    pmean = jnp.sum(probs, axis=1, keepdims=True) / _N
    cnt = sel1.astype(jnp.float32) + sel2.astype(jnp.float32)
    counts = jnp.sum(cnt, axis=1, keepdims=True)
    aux_ref[0, 0] = _E * jnp.sum((counts / _N) * pmean)
    acc = cnt
    d = 1
    while d < _N:
        z = jnp.zeros((_E, d), jnp.float32)
        acc = acc + jnp.concatenate([z, acc[:, :-d]], axis=1)
        d *= 2
    prefix = acc - cnt
    cpad = jnp.floor((counts + (_T - 1)) / _T) * _T
    o = cpad
    o = o + jnp.concatenate([jnp.zeros((1, 1), jnp.float32), o[:-1]], axis=0)
    o = o + jnp.concatenate([jnp.zeros((2, 1), jnp.float32), o[:-2]], axis=0)
    o = o + jnp.concatenate([jnp.zeros((4, 1), jnp.float32), o[:-4]], axis=0)
    off = o - cpad
    end = off + cpad
    slot = off + prefix
    d1 = jnp.sum(jnp.where(sel1, slot, 0.0), axis=0, keepdims=True)
    d2 = jnp.sum(jnp.where(sel2, slot, 0.0), axis=0, keepdims=True)
    destp_ref[...] = jnp.concatenate([d1, d2], axis=0).astype(jnp.int32)
    wp_ref[...] = jnp.concatenate([w1, w2], axis=0)
    sb = jax.lax.broadcasted_iota(jnp.int32, (1, 64), 1).astype(
        jnp.float32) * _T
    g = jnp.sum((sb >= end).astype(jnp.float32), axis=0, keepdims=True)
    gid_ref[...] = jnp.minimum(g, _E - 1).astype(jnp.int32)
    nb_ref[0, 0] = (jnp.sum(cpad) / _T).astype(jnp.int32)


def _sc_dispatch_body(destp_hbm, wp_hbm, x_hbm, xs_hbm, scale_hbm,
                      initv, tokv, idx128a, idx128b, wv256, src_sh, scale_sh,
                      idxv, rows0, rows1, gsem, wsem):
    cid = lax.axis_index("c")
    sid = lax.axis_index("s")
    ibase = sid * _IPS
    for c in range(_IPS // _L):
        initv[pl.ds(c * _L, _L)] = (
            lax.iota(jnp.int32, _L) + (ibase + c * _L)) % _N
    pltpu.sync_copy(initv, src_sh.at[pl.ds(ibase, _IPS)])
    plsc.subcore_barrier()
    pbase = sid * _PPS
    la = pltpu.async_copy(destp_hbm.at[pl.ds(pbase, 128)], idx128a, gsem)
    lb = pltpu.async_copy(destp_hbm.at[pl.ds(pbase + 128, 128)], idx128b,
                          gsem)
    lw = pltpu.async_copy(wp_hbm.at[pl.ds(pbase, _PPS)], wv256, wsem)
    for t in range(_PPS // _L):
        tokv[pl.ds(t * _L, _L)] = (
            lax.iota(jnp.int32, _L) + (pbase + t * _L)) % _N
    la.wait()
    lb.wait()
    lw.wait()
    s1 = pltpu.async_copy(tokv.at[pl.ds(0, 128)], src_sh.at[idx128a], gsem)
    s2 = pltpu.async_copy(tokv.at[pl.ds(128, 128)], src_sh.at[idx128b], gsem)
    s3 = pltpu.async_copy(wv256.at[pl.ds(0, 128)], scale_sh.at[idx128a],
                          wsem)
    s4 = pltpu.async_copy(wv256.at[pl.ds(128, 128)], scale_sh.at[idx128b],
                          wsem)
    s1.wait()
    s2.wait()
    s3.wait()
    s4.wait()
    plsc.subcore_barrier()

    @pl.when(jnp.logical_and(sid == 0, cid == 0))
    def _():
        pltpu.sync_copy(scale_sh, scale_hbm)

    wid = sid * _NC + cid
    tbase = wid * _RPT
    rows = (rows0, rows1)
    wb = [None, None]
    for j in range(_RPT // _GCH):
        b = j % 2
        if j >= 2:
            wb[b].wait()
        pltpu.sync_copy(src_sh.at[pl.ds(tbase + j * _GCH, _GCH)], idxv)
        pltpu.async_copy(x_hbm.at[idxv], rows[b], gsem).wait()
        wb[b] = pltpu.async_copy(
            rows[b], xs_hbm.at[pl.ds(tbase + j * _GCH, _GCH)], wsem)
    wb[0].wait()
    wb[1].wait()


def _ffn_body(gid_ref, nb_ref, xs_ref, sc_ref, wg_ref, wu_ref, wd_ref,
              ys_ref):
    @pl.when(pl.program_id(0) < nb_ref[0])
    def _():
        xb = xs_ref[...]
        g = jax.lax.dot_general(xb, wg_ref[0], (((1,), (1,)), ((), ())),
                                preferred_element_type=jnp.float32)
        u = jax.lax.dot_general(xb, wu_ref[0], (((1,), (1,)), ((), ())),
                                preferred_element_type=jnp.float32)
        h = (g * jax.nn.sigmoid(g)) * u
        y = jax.lax.dot_general(h, wd_ref[0], (((1,), (1,)), ((), ())),
                                preferred_element_type=jnp.float32)
        ys_ref[...] = sc_ref[...] * y


def _sc_combine_body(d0_hbm, d1_hbm, ys_hbm, out_hbm,
                     idxa, idxb, ra, rb, sem):
    cid = lax.axis_index("c")
    sid = lax.axis_index("s")
    wid = sid * _NC + cid
    base = wid * _TPT
    pltpu.sync_copy(d0_hbm.at[pl.ds(base, _TPT)], idxa)
    pltpu.sync_copy(d1_hbm.at[pl.ds(base, _TPT)], idxb)
    ca = pltpu.async_copy(ys_hbm.at[idxa], ra, sem)
    cb = pltpu.async_copy(ys_hbm.at[idxb], rb, sem)
    ca.wait()
    cb.wait()

    def rowbody(r, _):
        for c in range(_D // _L):
            sl = pl.ds(c * _L, _L)
            ra[r, sl] = ra[r, sl] + rb[r, sl]
        return 0

    half = _TPT // 2
    lax.fori_loop(0, half, rowbody, 0)
    wb = pltpu.async_copy(ra.at[pl.ds(0, half)],
                          out_hbm.at[pl.ds(base, half)], sem)
    lax.fori_loop(half, _TPT, rowbody, 0)
    pltpu.sync_copy(ra.at[pl.ds(half, half)],
                    out_hbm.at[pl.ds(base + half, half)])
    wb.wait()


def _sc_dispatch(destp_flat, wp_flat, flat):
    wrapped = functools.partial(
        pl.kernel,
        out_type=(
            jax.ShapeDtypeStruct((_PMAX, _D), jnp.float32),
            jax.ShapeDtypeStruct((_PMAX,), jnp.float32),
        ),
        mesh=plsc.VectorSubcoreMesh(core_axis_name="c", subcore_axis_name="s",
                                    num_cores=_NC, num_subcores=_NS),
        scratch_types=[
            pltpu.VMEM((_IPS,), jnp.int32),
            pltpu.VMEM((_PPS,), jnp.int32),
            pltpu.VMEM((128,), jnp.int32),
            pltpu.VMEM((128,), jnp.int32),
            pltpu.VMEM((_PPS,), jnp.float32),
            pltpu.VMEM_SHARED((_PMAX,), jnp.int32),
            pltpu.VMEM_SHARED((_PMAX,), jnp.float32),
            pltpu.VMEM((_GCH,), jnp.int32),
            pltpu.VMEM((_GCH, _D), jnp.float32),
            pltpu.VMEM((_GCH, _D), jnp.float32),
            pltpu.SemaphoreType.DMA,
            pltpu.SemaphoreType.DMA,
        ],
        compiler_params=pltpu.CompilerParams(needs_layout_passes=False),
    )(_sc_dispatch_body)
    return wrapped(destp_flat, wp_flat, flat)


def _sc_combine(d0, d1, ys):
    wrapped = functools.partial(
        pl.kernel,
        out_type=jax.ShapeDtypeStruct((_N, _D), jnp.float32),
        mesh=plsc.VectorSubcoreMesh(core_axis_name="c", subcore_axis_name="s",
                                    num_cores=_NC, num_subcores=_NS),
        scratch_types=[
            pltpu.VMEM((_TPT,), jnp.int32),
            pltpu.VMEM((_TPT,), jnp.int32),
            pltpu.VMEM((_TPT, _D), jnp.float32),
            pltpu.VMEM((_TPT, _D), jnp.float32),
            pltpu.SemaphoreType.DMA,
        ],
        compiler_params=pltpu.CompilerParams(needs_layout_passes=False),
    )(_sc_combine_body)
    return wrapped(d0, d1, ys)


@jax.jit
def kernel(x, gate_w, w_gate, w_up, w_down):
    flat = x.reshape(_N, _D)

    destp, wp, gid, nb, aux = pl.pallas_call(
        _router_body,
        out_shape=(
            jax.ShapeDtypeStruct((2, _N), jnp.int32),
            jax.ShapeDtypeStruct((2, _N), jnp.float32),
            jax.ShapeDtypeStruct((1, 64), jnp.int32),
            jax.ShapeDtypeStruct((1, 1), jnp.int32),
            jax.ShapeDtypeStruct((1, 1), jnp.float32),
        ),
        in_specs=[
            pl.BlockSpec(memory_space=pltpu.VMEM),
            pl.BlockSpec(memory_space=pltpu.VMEM),
        ],
        out_specs=(
            pl.BlockSpec(memory_space=pltpu.VMEM),
            pl.BlockSpec(memory_space=pltpu.VMEM),
            pl.BlockSpec(memory_space=pltpu.VMEM),
            pl.BlockSpec(memory_space=pltpu.SMEM),
            pl.BlockSpec(memory_space=pltpu.SMEM),
        ),
    )(flat, gate_w)

    xs, scale = _sc_dispatch(destp.reshape(2 * _N), wp.reshape(2 * _N), flat)
    scale = scale.reshape(_PMAX, 1)

    ys = pl.pallas_call(
        _ffn_body,
        grid_spec=pltpu.PrefetchScalarGridSpec(
            num_scalar_prefetch=2,
            grid=(_NB,),
            in_specs=[
                pl.BlockSpec((_T, _D), lambda b, gid, nb: (b, 0)),
                pl.BlockSpec((_T, 1), lambda b, gid, nb: (b, 0)),
                pl.BlockSpec((1, _FF, _D), lambda b, gid, nb: (gid[b], 0, 0)),
                pl.BlockSpec((1, _FF, _D), lambda b, gid, nb: (gid[b], 0, 0)),
                pl.BlockSpec((1, _D, _FF), lambda b, gid, nb: (gid[b], 0, 0)),
            ],
            out_specs=pl.BlockSpec((_T, _D), lambda b, gid, nb: (b, 0)),
        ),
        out_shape=jax.ShapeDtypeStruct((_PMAX, _D), jnp.float32),
    )(gid.reshape(64), nb.reshape(1), xs, scale, w_gate, w_up, w_down)

    out = _sc_combine(destp[0], destp[1], ys)
    return out.reshape(_B, _S, _D), aux.reshape(())
